# baseline (speedup 1.0000x reference)
import jax
import jax.numpy as jnp
from jax.experimental import pallas as pl
from jax.experimental.pallas import tpu as pltpu

SEG = 128
H = 2
C = 64
HC = H * C
TI = 64
JB = 512
SLOPE = 0.2


def _proj_body(x_ref, wcat_ref, sp_ref, xlt_ref, xl_ref, td_ref):
    t = jnp.dot(x_ref[...], wcat_ref[...],
                preferred_element_type=jnp.float32) + sp_ref[0:1, :]
    xl = t[:, 0:HC]
    xlt_ref[...] = xl.T
    xl_ref[...] = xl
    td_ref[...] = t[:, SEG:3 * SEG]


def _proj(x, wcat, sp):
    n = x.shape[0]
    fin = x.shape[1]
    return pl.pallas_call(
        _proj_body,
        out_shape=(jax.ShapeDtypeStruct((HC, n), jnp.float32),
                   jax.ShapeDtypeStruct((n, HC), jnp.float32),
                   jax.ShapeDtypeStruct((n, 2 * SEG), jnp.float32)),
        grid=(2,),
        in_specs=[
            pl.BlockSpec((n // 2, fin), lambda i: (i, 0)),
            pl.BlockSpec((fin, 3 * SEG), lambda i: (0, 0)),
            pl.BlockSpec((8, 3 * SEG), lambda i: (0, 0)),
        ],
        out_specs=(pl.BlockSpec((HC, n // 2), lambda i: (0, i)),
                   pl.BlockSpec((n // 2, HC), lambda i: (i, 0)),
                   pl.BlockSpec((n // 2, 2 * SEG), lambda i: (i, 0))),
        compiler_params=pltpu.CompilerParams(
            dimension_semantics=("parallel",)),
    )(x, wcat, sp)


def _make_attn_body(n, apply_prelu):
    def body(xlt_ref, xl_ref, td_ref, adj_ref, ea_ref, sp_ref, sps_ref,
             out_ref, s_ref):
        td = td_ref[...]
        xr = td[:, 0:HC]
        skip = td[:, SEG:SEG + C]
        lane_iota = jax.lax.broadcasted_iota(jnp.int32, (TI, HC), 1)

        s_ref[...] = jnp.zeros((H, TI, n), jnp.float32)

        def make_octet(h):
            def octet(o, carry):
                c0 = h * C + o * 8
                xl8 = xlt_ref[pl.ds(c0, 8), :]
                scal = []
                xr_cols = []
                for u in range(8):
                    c = c0 + u
                    scal.append((sps_ref[1, c], sps_ref[2, c],
                                 sps_ref[3, c], sps_ref[4, c]))
                    xr_cols.append(jnp.sum(
                        jnp.where(lane_iota == c, xr, 0.0),
                        axis=1, keepdims=True))
                for jb in range(n // JB):
                    j0 = jb * JB
                    acc = s_ref[h, :, j0:j0 + JB]
                    for u in range(8):
                        we0, we1, we2, af = scal[u]
                        v = (ea_ref[0, :, j0:j0 + JB] * we0
                             + ea_ref[1, :, j0:j0 + JB] * we1
                             + ea_ref[2, :, j0:j0 + JB] * we2
                             + xl8[u:u + 1, j0:j0 + JB]
                             + xr_cols[u])
                        acc = acc + jnp.maximum(v, SLOPE * v) * af
                    s_ref[h, :, j0:j0 + JB] = acc
                return carry
            return octet

        for h in range(H):
            jax.lax.fori_loop(0, C // 8, make_octet(h), 0)

        adj = adj_ref[...]
        sp = sp_ref[...]
        gb = sp[5:6, 0:C]
        pw = sp[6:7, 0:C]
        bias = jnp.where(adj > 0, 0.0, -1e30)
        acc = None
        for h in range(H):
            s = s_ref[h] + bias
            smax = jnp.max(s, axis=1, keepdims=True)
            p = jnp.exp(s - smax) * adj
            denom = jnp.sum(p, axis=1, keepdims=True) + 1e-16
            alpha = p * pl.reciprocal(denom, approx=True)
            xl_h = xl_ref[:, h * C:(h + 1) * C]
            d = jnp.dot(alpha, xl_h, preferred_element_type=jnp.float32)
            acc = d if acc is None else acc + d
        out = acc * (1.0 / H) + gb + skip
        if apply_prelu:
            out = jnp.where(out > 0, out, pw * out)
        out_ref[...] = out
    return body


def _attn(xlt, xl, td, adj, ea, sp, apply_prelu):
    n = adj.shape[0]
    return pl.pallas_call(
        _make_attn_body(n, apply_prelu),
        out_shape=jax.ShapeDtypeStruct((n, C), jnp.float32),
        grid=(n // TI,),
        in_specs=[
            pl.BlockSpec((HC, n), lambda i: (0, 0)),
            pl.BlockSpec((n, HC), lambda i: (0, 0)),
            pl.BlockSpec((TI, 2 * SEG), lambda i: (i, 0)),
            pl.BlockSpec((TI, n), lambda i: (i, 0)),
            pl.BlockSpec((3, TI, n), lambda i: (0, i, 0)),
            pl.BlockSpec((8, 3 * SEG), lambda i: (0, 0)),
            pl.BlockSpec(memory_space=pltpu.MemorySpace.SMEM),
        ],
        out_specs=pl.BlockSpec((TI, C), lambda i: (i, 0)),
        scratch_shapes=[pltpu.VMEM((H, TI, n), jnp.float32)],
        compiler_params=pltpu.CompilerParams(
            dimension_semantics=("parallel",)),
    )(xlt, xl, td, adj, ea, sp, sp)


def _layer(x, adj, ea, wcat, sp, apply_prelu):
    xlt, xl, td = _proj(x, wcat, sp)
    return _attn(xlt, xl, td, adj, ea, sp, apply_prelu)


def kernel(x, adj, ea, wcat1, sp1, wcat2, sp2):
    h1 = _layer(x, adj, ea, wcat1, sp1, False)
    return _layer(h1, adj, ea, wcat2, sp2, True)


# speedup vs baseline: 1.2468x; 1.2468x over previous
import jax
import jax.numpy as jnp
from jax.experimental import pallas as pl
from jax.experimental.pallas import tpu as pltpu

SEG = 128
H = 2
C = 64
HC = H * C
TI = 64
JB = 256
SLOPE = 0.2


def _proj_body(x_ref, wcat_ref, sp_ref, xlt_ref, xl_ref, td_ref):
    t = jnp.dot(x_ref[...], wcat_ref[...],
                preferred_element_type=jnp.float32) + sp_ref[0:1, :]
    xl = t[:, 0:HC]
    xlt_ref[...] = xl.T
    xl_ref[...] = xl
    td_ref[...] = t[:, SEG:3 * SEG]


def _proj(x, wcat, sp):
    n = x.shape[0]
    fin = x.shape[1]
    return pl.pallas_call(
        _proj_body,
        out_shape=(jax.ShapeDtypeStruct((HC, n), jnp.float32),
                   jax.ShapeDtypeStruct((n, HC), jnp.float32),
                   jax.ShapeDtypeStruct((n, 2 * SEG), jnp.float32)),
        grid=(2,),
        in_specs=[
            pl.BlockSpec((n // 2, fin), lambda i: (i, 0)),
            pl.BlockSpec((fin, 3 * SEG), lambda i: (0, 0)),
            pl.BlockSpec((8, 3 * SEG), lambda i: (0, 0)),
        ],
        out_specs=(pl.BlockSpec((HC, n // 2), lambda i: (0, i)),
                   pl.BlockSpec((n // 2, HC), lambda i: (i, 0)),
                   pl.BlockSpec((n // 2, 2 * SEG), lambda i: (i, 0))),
        compiler_params=pltpu.CompilerParams(
            dimension_semantics=("parallel",)),
    )(x, wcat, sp)


def _make_attn_body(n, apply_prelu):
    def body(xlt_ref, xl_ref, td_ref, adj_ref, ea_ref, sp_ref, sps_ref,
             out_ref, s_ref):
        td = td_ref[...]
        xr = td[:, 0:HC]
        skip = td[:, SEG:SEG + C]
        lane_iota = jax.lax.broadcasted_iota(jnp.int32, (TI, HC), 1)

        sp_v = sp_ref[...]
        we_v = sp_v[1:4, 0:HC]
        attf = sp_v[4:5, 0:HC]
        row_iota = jax.lax.broadcasted_iota(jnp.int32, (1, HC), 1) // C
        for h in range(H):
            attf_h = jnp.where(row_iota == h, attf, 0.0)
            al = jnp.dot(attf_h, xlt_ref[...],
                         preferred_element_type=jnp.float32)
            ar = jnp.sum(xr * attf_h, axis=1, keepdims=True)
            ce = jnp.sum(we_v * attf_h, axis=1, keepdims=True)
            s_lin = (ea_ref[0] * ce[0, 0] + ea_ref[1] * ce[1, 0]
                     + ea_ref[2] * ce[2, 0] + al + ar)
            s_ref[h] = 1.5 * s_lin

        def make_octet(h):
            def octet(o, carry):
                c0 = h * C + o * 8
                xl8 = xlt_ref[pl.ds(c0, 8), :]
                scal = []
                xr_cols = []
                for u in range(8):
                    c = c0 + u
                    scal.append((sps_ref[1, c], sps_ref[2, c],
                                 sps_ref[3, c], sps_ref[4, c]))
                    xr_cols.append(jnp.sum(
                        jnp.where(lane_iota == c, xr, 0.0),
                        axis=1, keepdims=True))
                for jb in range(n // JB):
                    j0 = jb * JB
                    acc = s_ref[h, :, j0:j0 + JB]
                    for u in range(8):
                        we0, we1, we2, af = scal[u]
                        v = (ea_ref[0, :, j0:j0 + JB] * we0
                             + ea_ref[1, :, j0:j0 + JB] * we1
                             + ea_ref[2, :, j0:j0 + JB] * we2
                             + xl8[u:u + 1, j0:j0 + JB]
                             + xr_cols[u])
                        acc = acc + jnp.abs(v) * af
                    s_ref[h, :, j0:j0 + JB] = acc
                return carry
            return octet

        for h in range(H):
            jax.lax.fori_loop(0, C // 8, make_octet(h), 0)

        adj = adj_ref[...]
        sp = sp_ref[...]
        gb = sp[5:6, 0:C]
        pw = sp[6:7, 0:C]
        bias = jnp.where(adj > 0, 0.0, -1e30)
        acc = None
        for h in range(H):
            s = s_ref[h] * 0.4 + bias
            smax = jnp.max(s, axis=1, keepdims=True)
            p = jnp.exp(s - smax) * adj
            denom = jnp.sum(p, axis=1, keepdims=True) + 1e-16
            alpha = p * pl.reciprocal(denom, approx=True)
            xl_h = xl_ref[:, h * C:(h + 1) * C]
            d = jnp.dot(alpha, xl_h, preferred_element_type=jnp.float32)
            acc = d if acc is None else acc + d
        out = acc * (1.0 / H) + gb + skip
        if apply_prelu:
            out = jnp.where(out > 0, out, pw * out)
        out_ref[...] = out
    return body


def _attn(xlt, xl, td, adj, ea, sp, apply_prelu):
    n = adj.shape[0]
    return pl.pallas_call(
        _make_attn_body(n, apply_prelu),
        out_shape=jax.ShapeDtypeStruct((n, C), jnp.float32),
        grid=(n // TI,),
        in_specs=[
            pl.BlockSpec((HC, n), lambda i: (0, 0)),
            pl.BlockSpec((n, HC), lambda i: (0, 0)),
            pl.BlockSpec((TI, 2 * SEG), lambda i: (i, 0)),
            pl.BlockSpec((TI, n), lambda i: (i, 0)),
            pl.BlockSpec((3, TI, n), lambda i: (0, i, 0)),
            pl.BlockSpec((8, 3 * SEG), lambda i: (0, 0)),
            pl.BlockSpec(memory_space=pltpu.MemorySpace.SMEM),
        ],
        out_specs=pl.BlockSpec((TI, C), lambda i: (i, 0)),
        scratch_shapes=[pltpu.VMEM((H, TI, n), jnp.float32)],
        compiler_params=pltpu.CompilerParams(
            dimension_semantics=("parallel",)),
    )(xlt, xl, td, adj, ea, sp, sp)


def _layer(x, adj, ea, wcat, sp, apply_prelu):
    xlt, xl, td = _proj(x, wcat, sp)
    return _attn(xlt, xl, td, adj, ea, sp, apply_prelu)


def kernel(x, adj, ea, wcat1, sp1, wcat2, sp2):
    h1 = _layer(x, adj, ea, wcat1, sp1, False)
    return _layer(h1, adj, ea, wcat2, sp2, True)
